# baseline (speedup 1.0000x reference)
import jax
import jax.numpy as jnp
from jax.experimental import pallas as pl

_STUFF_THRESH = 10
_INST_BASE = 11


def _body(segs_ref, masks_ref, out_ref):
    num_gt = masks_ref.shape[0]
    seg = segs_ref[0]
    best = jnp.full(seg.shape, -1, jnp.int32)
    for i in range(num_gt):
        best = jnp.where(masks_ref[i] != 0.0, i, best)
    stuff = jnp.where((seg <= _STUFF_THRESH) | (seg >= 255), seg, 255)
    out_ref[0] = jnp.where(best >= 0, best + _INST_BASE, stuff)


def kernel(gt_segs, gt_masks):
    _, H, W = gt_segs.shape
    num_gt = gt_masks.shape[0]
    RB = 8
    grid = (H // RB,)
    out = pl.pallas_call(
        _body,
        grid=grid,
        in_specs=[
            pl.BlockSpec((1, RB, W), lambda i: (0, i, 0)),
            pl.BlockSpec((num_gt, RB, W), lambda i: (0, i, 0)),
        ],
        out_specs=pl.BlockSpec((1, RB, W), lambda i: (0, i, 0)),
        out_shape=jax.ShapeDtypeStruct(gt_segs.shape, gt_segs.dtype),
    )(gt_segs, gt_masks)
    return out
